# baseline (speedup 1.0000x reference)
import functools

import jax
import jax.numpy as jnp
from jax import lax
from jax.experimental import pallas as pl
from jax.experimental.pallas import tpu as pltpu
from jax.experimental.pallas import tpu_sc as plsc

N = 10000
E = 320000
D = 128
L = 3

NC = 2
NS = 16
NW = NC * NS
C = 128
NCH_DEG = 80
PE = NW * NCH_DEG * C
NCH0 = 128
NCH1 = 32
NCHMAX = NCH0
NBUF = 2
SLOTS = 8
NPAD = 10240
RPT = NPAD // NS


_MESH = plsc.VectorSubcoreMesh(core_axis_name="c", subcore_axis_name="s")


def _zero_vmem_rows(buf, nrows, ncols):
    z = jnp.zeros((16,), jnp.float32)

    def row(i, _):
        for j in range(ncols // 16):
            buf[i, pl.ds(j * 16, 16)] = z
        return 0

    lax.fori_loop(0, nrows, row, 0)


@functools.partial(
    pl.kernel,
    out_type=jax.ShapeDtypeStruct((NC, NPAD, D), jnp.float32),
    mesh=_MESH,
    scratch_types=[
        pltpu.VMEM((NCH_DEG, C), jnp.int32),
        pltpu.VMEM((C, D), jnp.float32),
        pltpu.VMEM_SHARED((NPAD, D), jnp.float32),
        pltpu.SemaphoreType.DMA,
    ],
)
def _sc_degree(dst_hbm, out_hbm, dst_v, ones_v, deg_sh, sem):
    c = lax.axis_index("c")
    s = lax.axis_index("s")
    w = c * NS + s
    pltpu.sync_copy(dst_hbm.at[w], dst_v)
    _zero_vmem_rows(ones_v, C, D)
    base = s * RPT
    nfull = RPT // C
    for r in range(nfull):
        pltpu.sync_copy(ones_v, deg_sh.at[pl.ds(base + r * C, C)])
    rem = RPT - nfull * C
    if rem:
        pltpu.sync_copy(ones_v.at[pl.ds(0, rem)],
                        deg_sh.at[pl.ds(base + nfull * C, rem)])
    one = jnp.ones((16,), jnp.float32)

    def fill(i, _):
        for j in range(D // 16):
            ones_v[i, pl.ds(j * 16, 16)] = one
        return 0

    lax.fori_loop(0, C, fill, 0)
    plsc.subcore_barrier()

    def chunk(g, _):
        pltpu.async_copy(ones_v, deg_sh.at[dst_v.at[g]], sem, add=True)
        return 0

    lax.fori_loop(0, NCH_DEG, chunk, 0)

    def drain(g, _):
        pltpu.make_async_copy(ones_v, deg_sh.at[dst_v.at[g]], sem).wait()
        return 0

    lax.fori_loop(0, NCH_DEG, drain, 0)
    plsc.subcore_barrier()
    pltpu.sync_copy(deg_sh.at[pl.ds(s * RPT, RPT)],
                    out_hbm.at[c, pl.ds(s * RPT, RPT)])


@functools.partial(
    pl.kernel,
    out_type=jax.ShapeDtypeStruct((NC, NPAD, D), jnp.float32),
    mesh=_MESH,
    scratch_types=[
        pltpu.VMEM((SLOTS, C), jnp.int32),
        pltpu.VMEM((SLOTS, C), jnp.int32),
        pltpu.VMEM((NBUF, C, D), jnp.float32),
        pltpu.VMEM_SHARED((NPAD, D), jnp.float32),
        pltpu.SemaphoreType.DMA((SLOTS,)),
        pltpu.SemaphoreType.DMA((SLOTS,)),
        pltpu.SemaphoreType.DMA((NBUF,)),
        pltpu.SemaphoreType.DMA((NBUF,)),
    ],
)
def _sc_scatter(y_hbm, src_hbm, dst_hbm, out_hbm, src_v, dst_v, rows_v,
                acc_sh, isem, dsem, gsem, ssem):
    c = lax.axis_index("c")
    s = lax.axis_index("s")
    w = c * NS + s
    nch = jnp.where(c == 0, NCH0, NCH1)
    trips = nch // SLOTS

    def src_dma(g, sl):
        pltpu.async_copy(src_hbm.at[w, g], src_v.at[sl], isem.at[sl])

    def wait_src(g, sl):
        pltpu.make_async_copy(src_hbm.at[w, g], src_v.at[sl],
                              isem.at[sl]).wait()

    def dst_dma(g, sl):
        pltpu.async_copy(dst_hbm.at[w, g], dst_v.at[sl], dsem.at[sl])

    def wait_dst(g, sl):
        pltpu.make_async_copy(dst_hbm.at[w, g], dst_v.at[sl],
                              dsem.at[sl]).wait()

    def gather(sl, b):
        pltpu.async_copy(y_hbm.at[src_v.at[sl]], rows_v.at[b], gsem.at[b])

    def drain_gather(sl, b):
        pltpu.make_async_copy(y_hbm.at[src_v.at[sl]], rows_v.at[b],
                              gsem.at[b]).wait()

    def scatter(sl, b):
        pltpu.async_copy(rows_v.at[b], acc_sh.at[dst_v.at[sl]], ssem.at[b],
                         add=True)

    def drain_scatter(sl, b):
        pltpu.make_async_copy(rows_v.at[b], acc_sh.at[dst_v.at[sl]],
                              ssem.at[b]).wait()

    zb = rows_v.at[0]
    _zero_vmem_rows(zb, C, D)
    base = s * RPT
    for r in range(RPT // C):
        pltpu.sync_copy(zb, acc_sh.at[pl.ds(base + r * C, C)])
    for g0 in range(SLOTS - 2):
        src_dma(g0, g0)
        dst_dma(g0, g0)
    wait_src(0, 0)
    gather(0, 0)
    plsc.subcore_barrier()

    t_last = trips - 1

    def outer(t, _):
        for u in range(SLOTS):
            g = t * SLOTS + u
            b = u % NBUF
            bn = (u + 1) % NBUF
            sl1 = (u + 1) % SLOTS
            sl6 = (u + 6) % SLOTS
            sl7 = (u + 7) % SLOTS

            if u == 0:
                @pl.when(t > 0)
                def _():
                    drain_scatter(sl7, bn)
            else:
                drain_scatter(sl7, bn)
            if u < 2:
                src_dma(g + 6, sl6)
                dst_dma(g + 6, sl6)
            else:
                @pl.when(t < t_last)
                def _():
                    src_dma(g + 6, sl6)
                    dst_dma(g + 6, sl6)
            if u < SLOTS - 1:
                wait_src(g + 1, sl1)
                gather(sl1, bn)
            else:
                @pl.when(t < t_last)
                def _():
                    wait_src(g + 1, sl1)
                    gather(sl1, bn)
            drain_gather(u, b)
            wait_dst(g, u)
            scatter(u, b)
        return 0

    lax.fori_loop(0, trips, outer, 0)
    drain_scatter((NCH0 - 1) % SLOTS, 1)
    plsc.subcore_barrier()
    pltpu.sync_copy(acc_sh.at[pl.ds(s * RPT, RPT)],
                    out_hbm.at[c, pl.ds(s * RPT, RPT)])


BM = 1024
GRID = (N + BM - 1) // BM


def _dinv_of(degp_ref):
    deg = degp_ref[0, :, 0:1] + degp_ref[1, :, 0:1] + 1.0
    return lax.rsqrt(deg)


def _pre_body(x_ref, wp_ref, bp_ref, wc0_ref, degp_ref, h_ref, y_ref):
    h = jnp.dot(x_ref[...], wp_ref[...],
                preferred_element_type=jnp.float32) + bp_ref[...]
    h_ref[...] = h
    y_ref[...] = jnp.dot(h * _dinv_of(degp_ref), wc0_ref[...],
                         preferred_element_type=jnp.float32)


def _make_layer_body(residual, has_next):
    def body(parts_ref, y_ref, hprev_ref, degp_ref, bc_ref, g_ref, b_ref,
             *rest):
        if has_next:
            wc_ref, h_ref, ynext_ref = rest
        else:
            wc_ref = None
            (h_ref,) = rest
        dinv = _dinv_of(degp_ref)
        acc = parts_ref[0] + parts_ref[1] + y_ref[...]
        t = acc * dinv + bc_ref[...]
        mu = jnp.mean(t, axis=-1, keepdims=True)
        tc = t - mu
        var = jnp.mean(tc * tc, axis=-1, keepdims=True)
        t = tc * lax.rsqrt(var + 1e-5) * g_ref[...] + b_ref[...]
        t = jnp.maximum(t, 0.0)
        if residual:
            t = t + hprev_ref[...]
        h_ref[...] = t
        if has_next:
            ynext_ref[...] = jnp.dot(t * dinv, wc_ref[...],
                                     preferred_element_type=jnp.float32)

    return body


def _row_spec():
    return pl.BlockSpec((BM, D), lambda i: (i, 0))


def _full_spec(shape):
    nd = len(shape)
    return pl.BlockSpec(shape, lambda i: (0,) * nd)


def _degp_spec():
    return pl.BlockSpec((2, BM, D), lambda i: (0, i, 0))


def _tc_pre(x, Wp, bp, Wc0, degp):
    return pl.pallas_call(
        _pre_body,
        grid=(GRID,),
        in_specs=[
            _row_spec(),
            _full_spec((D, D)),
            _full_spec((1, D)),
            _full_spec((D, D)),
            _degp_spec(),
        ],
        out_specs=[_row_spec(), _row_spec()],
        out_shape=[
            jax.ShapeDtypeStruct((N, D), jnp.float32),
            jax.ShapeDtypeStruct((N, D), jnp.float32),
        ],
    )(x, Wp, bp.reshape(1, D), Wc0, degp)


def _tc_layer(parts, y, hprev, degp, bc_i, g_i, b_i, wc_next, residual):
    has_next = wc_next is not None
    ins = [parts, y, hprev, degp, bc_i.reshape(1, D), g_i.reshape(1, D),
           b_i.reshape(1, D)]
    in_specs = [
        pl.BlockSpec((2, BM, D), lambda i: (0, i, 0)),
        _row_spec(),
        _row_spec(),
        _degp_spec(),
        _full_spec((1, D)),
        _full_spec((1, D)),
        _full_spec((1, D)),
    ]
    if has_next:
        ins.append(wc_next)
        in_specs.append(_full_spec((D, D)))
        out_specs = [_row_spec(), _row_spec()]
        out_shape = [
            jax.ShapeDtypeStruct((N, D), jnp.float32),
            jax.ShapeDtypeStruct((N, D), jnp.float32),
        ]
    else:
        out_specs = [_row_spec()]
        out_shape = [jax.ShapeDtypeStruct((N, D), jnp.float32)]
    res = pl.pallas_call(
        _make_layer_body(residual, has_next),
        grid=(GRID,),
        in_specs=in_specs,
        out_specs=out_specs,
        out_shape=out_shape,
    )(*ins)
    return res if has_next else (res[0], None)


def kernel(x, edge_index, Wp, bp, Wc, bc, gamma, beta):
    src = edge_index[0]
    dst = edge_index[1]
    pad = PE - E
    src_all = jnp.concatenate([src, jnp.zeros((pad,), jnp.int32)])
    dst_all = jnp.concatenate([dst, jnp.full((pad,), N, jnp.int32)])
    dstp_even = dst_all.reshape(NW, NCH_DEG, C)
    e1 = NS * NCH1 * C
    pad_w = ((0, 0), (0, NCHMAX - NCH1), (0, 0))
    src1 = jnp.pad(src_all[:e1].reshape(NS, NCH1, C), pad_w)
    dst1 = jnp.pad(dst_all[:e1].reshape(NS, NCH1, C), pad_w,
                   constant_values=N)
    src0 = src_all[e1:].reshape(NS, NCH0, C)
    dst0 = dst_all[e1:].reshape(NS, NCH0, C)
    srcp = jnp.concatenate([src0, src1], axis=0)
    dstp = jnp.concatenate([dst0, dst1], axis=0)

    degp = _sc_degree(dstp_even)
    h, y = _tc_pre(x, Wp, bp, Wc[0], degp)
    for i in range(L):
        parts = _sc_scatter(y, srcp, dstp)
        wc_next = Wc[i + 1] if i < L - 1 else None
        h, y = _tc_layer(parts, y, h, degp, bc[i], gamma[i], beta[i],
                         wc_next, residual=(i > 0))
    return h


# speedup vs baseline: 1.0764x; 1.0764x over previous
import functools

import jax
import jax.numpy as jnp
from jax import lax
from jax.experimental import pallas as pl
from jax.experimental.pallas import tpu as pltpu
from jax.experimental.pallas import tpu_sc as plsc

N = 10000
E = 320000
D = 128
L = 3

NC = 2
NS = 16
NW = NC * NS
C = 128
NCH = 80
NCH_DEG = NCH
PE = NW * NCH * C
NBUF = 2
SLOTS = 8
NPAD = 10240
RPT = NPAD // NS


_MESH = plsc.VectorSubcoreMesh(core_axis_name="c", subcore_axis_name="s")


def _zero_vmem_rows(buf, nrows, ncols):
    z = jnp.zeros((16,), jnp.float32)

    def row(i, _):
        for j in range(ncols // 16):
            buf[i, pl.ds(j * 16, 16)] = z
        return 0

    lax.fori_loop(0, nrows, row, 0)


@functools.partial(
    pl.kernel,
    out_type=jax.ShapeDtypeStruct((NC, NPAD, D), jnp.float32),
    mesh=_MESH,
    scratch_types=[
        pltpu.VMEM((NCH_DEG, C), jnp.int32),
        pltpu.VMEM((C, D), jnp.float32),
        pltpu.VMEM_SHARED((NPAD, D), jnp.float32),
        pltpu.SemaphoreType.DMA,
    ],
)
def _sc_degree(dst_hbm, out_hbm, dst_v, ones_v, deg_sh, sem):
    c = lax.axis_index("c")
    s = lax.axis_index("s")
    w = c * NS + s
    pltpu.sync_copy(dst_hbm.at[w], dst_v)
    _zero_vmem_rows(ones_v, C, D)
    base = s * RPT
    nfull = RPT // C
    for r in range(nfull):
        pltpu.sync_copy(ones_v, deg_sh.at[pl.ds(base + r * C, C)])
    rem = RPT - nfull * C
    if rem:
        pltpu.sync_copy(ones_v.at[pl.ds(0, rem)],
                        deg_sh.at[pl.ds(base + nfull * C, rem)])
    one = jnp.ones((16,), jnp.float32)

    def fill(i, _):
        for j in range(D // 16):
            ones_v[i, pl.ds(j * 16, 16)] = one
        return 0

    lax.fori_loop(0, C, fill, 0)
    plsc.subcore_barrier()

    def chunk(g, _):
        pltpu.async_copy(ones_v, deg_sh.at[dst_v.at[g]], sem, add=True)
        return 0

    lax.fori_loop(0, NCH_DEG, chunk, 0)

    def drain(g, _):
        pltpu.make_async_copy(ones_v, deg_sh.at[dst_v.at[g]], sem).wait()
        return 0

    lax.fori_loop(0, NCH_DEG, drain, 0)
    plsc.subcore_barrier()
    pltpu.sync_copy(deg_sh.at[pl.ds(s * RPT, RPT)],
                    out_hbm.at[c, pl.ds(s * RPT, RPT)])


@functools.partial(
    pl.kernel,
    out_type=jax.ShapeDtypeStruct((NC, NPAD, D), jnp.float32),
    mesh=_MESH,
    scratch_types=[
        pltpu.VMEM((SLOTS, C), jnp.int32),
        pltpu.VMEM((SLOTS, C), jnp.int32),
        pltpu.VMEM((NBUF, C, D), jnp.float32),
        pltpu.VMEM_SHARED((NPAD, D), jnp.float32),
        pltpu.SemaphoreType.DMA((SLOTS,)),
        pltpu.SemaphoreType.DMA((SLOTS,)),
        pltpu.SemaphoreType.DMA((NBUF,)),
        pltpu.SemaphoreType.DMA((NBUF,)),
    ],
)
def _sc_scatter(y_hbm, src_hbm, dst_hbm, out_hbm, src_v, dst_v, rows_v,
                acc_sh, isem, dsem, gsem, ssem):
    c = lax.axis_index("c")
    s = lax.axis_index("s")
    w = c * NS + s
    trips = NCH // SLOTS

    def src_dma(g, sl):
        pltpu.async_copy(src_hbm.at[w, g], src_v.at[sl], isem.at[sl])

    def wait_src(g, sl):
        pltpu.make_async_copy(src_hbm.at[w, g], src_v.at[sl],
                              isem.at[sl]).wait()

    def dst_dma(g, sl):
        pltpu.async_copy(dst_hbm.at[w, g], dst_v.at[sl], dsem.at[sl])

    def wait_dst(g, sl):
        pltpu.make_async_copy(dst_hbm.at[w, g], dst_v.at[sl],
                              dsem.at[sl]).wait()

    def gather(sl, b):
        pltpu.async_copy(y_hbm.at[src_v.at[sl]], rows_v.at[b], gsem.at[b])

    def drain_gather(sl, b):
        pltpu.make_async_copy(y_hbm.at[src_v.at[sl]], rows_v.at[b],
                              gsem.at[b]).wait()

    def scatter(sl, b):
        pltpu.async_copy(rows_v.at[b], acc_sh.at[dst_v.at[sl]], ssem.at[b],
                         add=True)

    def drain_scatter(sl, b):
        pltpu.make_async_copy(rows_v.at[b], acc_sh.at[dst_v.at[sl]],
                              ssem.at[b]).wait()

    zb = rows_v.at[0]
    _zero_vmem_rows(zb, C, D)
    base = s * RPT
    for r in range(RPT // C):
        pltpu.sync_copy(zb, acc_sh.at[pl.ds(base + r * C, C)])
    for g0 in range(SLOTS - 2):
        src_dma(g0, g0)
        dst_dma(g0, g0)
    wait_src(0, 0)
    gather(0, 0)
    plsc.subcore_barrier()

    t_last = trips - 1

    def outer(t, _):
        for u in range(SLOTS):
            g = t * SLOTS + u
            b = u % NBUF
            bn = (u + 1) % NBUF
            sl1 = (u + 1) % SLOTS
            sl6 = (u + 6) % SLOTS
            sl7 = (u + 7) % SLOTS

            if u == 0:
                @pl.when(t > 0)
                def _():
                    drain_scatter(sl7, bn)
            else:
                drain_scatter(sl7, bn)
            if u < 2:
                src_dma(g + 6, sl6)
                dst_dma(g + 6, sl6)
            else:
                @pl.when(t < t_last)
                def _():
                    src_dma(g + 6, sl6)
                    dst_dma(g + 6, sl6)
            if u < SLOTS - 1:
                wait_src(g + 1, sl1)
                gather(sl1, bn)
            else:
                @pl.when(t < t_last)
                def _():
                    wait_src(g + 1, sl1)
                    gather(sl1, bn)
            drain_gather(u, b)
            wait_dst(g, u)
            scatter(u, b)
        return 0

    lax.fori_loop(0, trips, outer, 0)
    drain_scatter((NCH - 1) % SLOTS, (NCH - 1) % NBUF)
    plsc.subcore_barrier()
    pltpu.sync_copy(acc_sh.at[pl.ds(s * RPT, RPT)],
                    out_hbm.at[c, pl.ds(s * RPT, RPT)])


BM = 1024
GRID = (N + BM - 1) // BM


def _dinv_of(degp_ref):
    deg = degp_ref[0, :, 0:1] + degp_ref[1, :, 0:1] + 1.0
    return lax.rsqrt(deg)


def _pre_body(x_ref, wp_ref, bp_ref, wc0_ref, degp_ref, h_ref, y_ref):
    h = jnp.dot(x_ref[...], wp_ref[...],
                preferred_element_type=jnp.float32) + bp_ref[...]
    h_ref[...] = h
    y_ref[...] = jnp.dot(h * _dinv_of(degp_ref), wc0_ref[...],
                         preferred_element_type=jnp.float32)


def _make_layer_body(residual, has_next):
    def body(parts_ref, y_ref, hprev_ref, degp_ref, bc_ref, g_ref, b_ref,
             *rest):
        if has_next:
            wc_ref, h_ref, ynext_ref = rest
        else:
            wc_ref = None
            (h_ref,) = rest
        dinv = _dinv_of(degp_ref)
        acc = parts_ref[0] + parts_ref[1] + y_ref[...]
        t = acc * dinv + bc_ref[...]
        mu = jnp.mean(t, axis=-1, keepdims=True)
        tc = t - mu
        var = jnp.mean(tc * tc, axis=-1, keepdims=True)
        t = tc * lax.rsqrt(var + 1e-5) * g_ref[...] + b_ref[...]
        t = jnp.maximum(t, 0.0)
        if residual:
            t = t + hprev_ref[...]
        h_ref[...] = t
        if has_next:
            ynext_ref[...] = jnp.dot(t * dinv, wc_ref[...],
                                     preferred_element_type=jnp.float32)

    return body


def _row_spec():
    return pl.BlockSpec((BM, D), lambda i: (i, 0))


def _full_spec(shape):
    nd = len(shape)
    return pl.BlockSpec(shape, lambda i: (0,) * nd)


def _degp_spec():
    return pl.BlockSpec((2, BM, D), lambda i: (0, i, 0))


def _tc_pre(x, Wp, bp, Wc0, degp):
    return pl.pallas_call(
        _pre_body,
        grid=(GRID,),
        in_specs=[
            _row_spec(),
            _full_spec((D, D)),
            _full_spec((1, D)),
            _full_spec((D, D)),
            _degp_spec(),
        ],
        out_specs=[_row_spec(), _row_spec()],
        out_shape=[
            jax.ShapeDtypeStruct((N, D), jnp.float32),
            jax.ShapeDtypeStruct((N, D), jnp.float32),
        ],
    )(x, Wp, bp.reshape(1, D), Wc0, degp)


def _tc_layer(parts, y, hprev, degp, bc_i, g_i, b_i, wc_next, residual):
    has_next = wc_next is not None
    ins = [parts, y, hprev, degp, bc_i.reshape(1, D), g_i.reshape(1, D),
           b_i.reshape(1, D)]
    in_specs = [
        pl.BlockSpec((2, BM, D), lambda i: (0, i, 0)),
        _row_spec(),
        _row_spec(),
        _degp_spec(),
        _full_spec((1, D)),
        _full_spec((1, D)),
        _full_spec((1, D)),
    ]
    if has_next:
        ins.append(wc_next)
        in_specs.append(_full_spec((D, D)))
        out_specs = [_row_spec(), _row_spec()]
        out_shape = [
            jax.ShapeDtypeStruct((N, D), jnp.float32),
            jax.ShapeDtypeStruct((N, D), jnp.float32),
        ]
    else:
        out_specs = [_row_spec()]
        out_shape = [jax.ShapeDtypeStruct((N, D), jnp.float32)]
    res = pl.pallas_call(
        _make_layer_body(residual, has_next),
        grid=(GRID,),
        in_specs=in_specs,
        out_specs=out_specs,
        out_shape=out_shape,
    )(*ins)
    return res if has_next else (res[0], None)


def kernel(x, edge_index, Wp, bp, Wc, bc, gamma, beta):
    src = edge_index[0]
    dst = edge_index[1]
    pad = PE - E
    pad_dst = N + jnp.arange(pad, dtype=jnp.int32) % (NPAD - N)
    srcp = jnp.concatenate([src, jnp.zeros((pad,), jnp.int32)])
    dstp = jnp.concatenate([dst, pad_dst])
    srcp = srcp.reshape(NW, NCH, C)
    dstp = dstp.reshape(NW, NCH, C)

    degp = _sc_degree(dstp)
    h, y = _tc_pre(x, Wp, bp, Wc[0], degp)
    for i in range(L):
        parts = _sc_scatter(y, srcp, dstp)
        wc_next = Wc[i + 1] if i < L - 1 else None
        h, y = _tc_layer(parts, y, h, degp, bc[i], gamma[i], beta[i],
                         wc_next, residual=(i > 0))
    return h


# speedup vs baseline: 3.3428x; 3.1057x over previous
import functools

import jax
import jax.numpy as jnp
from jax import lax
from jax.experimental import pallas as pl
from jax.experimental.pallas import tpu as pltpu
from jax.experimental.pallas import tpu_sc as plsc

N = 10000
E = 320000
D = 128
L = 3

NC = 2
NS = 16
NW = NC * NS
C = 128
NCH = 80
NCH_DEG = NCH
PE = NW * NCH * C
NBUF = 2
SLOTS = 8
NPAD = 10240
RPT = NPAD // NS


_MESH = plsc.VectorSubcoreMesh(core_axis_name="c", subcore_axis_name="s")


def _zero_vmem_rows(buf, nrows, ncols):
    z = jnp.zeros((16,), jnp.float32)

    def row(i, _):
        for j in range(ncols // 16):
            buf[i, pl.ds(j * 16, 16)] = z
        return 0

    lax.fori_loop(0, nrows, row, 0)


@functools.partial(
    pl.kernel,
    out_type=jax.ShapeDtypeStruct((NC, NPAD, D), jnp.float32),
    mesh=_MESH,
    scratch_types=[
        pltpu.VMEM((NCH_DEG, C), jnp.int32),
        pltpu.VMEM((C, D), jnp.float32),
        pltpu.VMEM_SHARED((NPAD, D), jnp.float32),
        pltpu.SemaphoreType.DMA,
    ],
)
def _sc_degree(dst_hbm, out_hbm, dst_v, ones_v, deg_sh, sem):
    c = lax.axis_index("c")
    s = lax.axis_index("s")
    w = c * NS + s
    pltpu.sync_copy(dst_hbm.at[w], dst_v)
    _zero_vmem_rows(ones_v, C, D)
    base = s * RPT
    nfull = RPT // C
    for r in range(nfull):
        pltpu.sync_copy(ones_v, deg_sh.at[pl.ds(base + r * C, C)])
    rem = RPT - nfull * C
    if rem:
        pltpu.sync_copy(ones_v.at[pl.ds(0, rem)],
                        deg_sh.at[pl.ds(base + nfull * C, rem)])
    one = jnp.ones((16,), jnp.float32)

    def fill(i, _):
        for j in range(D // 16):
            ones_v[i, pl.ds(j * 16, 16)] = one
        return 0

    lax.fori_loop(0, C, fill, 0)
    plsc.subcore_barrier()

    def chunk(g, _):
        pltpu.async_copy(ones_v, deg_sh.at[dst_v.at[g]], sem, add=True)
        return 0

    lax.fori_loop(0, NCH_DEG, chunk, 0)

    def drain(g, _):
        pltpu.make_async_copy(ones_v, deg_sh.at[dst_v.at[g]], sem).wait()
        return 0

    lax.fori_loop(0, NCH_DEG, drain, 0)
    plsc.subcore_barrier()
    pltpu.sync_copy(deg_sh.at[pl.ds(s * RPT, RPT)],
                    out_hbm.at[c, pl.ds(s * RPT, RPT)])


@functools.partial(
    pl.kernel,
    out_type=jax.ShapeDtypeStruct((NC, NPAD, D), jnp.float32),
    mesh=_MESH,
    scratch_types=[
        pltpu.VMEM((SLOTS, C), jnp.int32),
        pltpu.VMEM((SLOTS, C), jnp.int32),
        pltpu.VMEM((NBUF, C, D), jnp.float32),
        pltpu.VMEM_SHARED((NPAD, D), jnp.float32),
        pltpu.SemaphoreType.DMA((SLOTS,)),
        pltpu.SemaphoreType.DMA((SLOTS,)),
        pltpu.SemaphoreType.DMA((NBUF,)),
        pltpu.SemaphoreType.DMA((NBUF,)),
    ],
)
def _sc_scatter(y_hbm, src_hbm, dst_hbm, out_hbm, src_v, dst_v, rows_v,
                acc_sh, isem, dsem, gsem, ssem):
    c = lax.axis_index("c")
    s = lax.axis_index("s")
    w = c * NS + s
    trips = NCH // SLOTS

    def src_dma(g, sl):
        pltpu.async_copy(src_hbm.at[w, g], src_v.at[sl], isem.at[sl])

    def wait_src(g, sl):
        pltpu.make_async_copy(src_hbm.at[w, g], src_v.at[sl],
                              isem.at[sl]).wait()

    def dst_dma(g, sl):
        pltpu.async_copy(dst_hbm.at[w, g], dst_v.at[sl], dsem.at[sl])

    def wait_dst(g, sl):
        pltpu.make_async_copy(dst_hbm.at[w, g], dst_v.at[sl],
                              dsem.at[sl]).wait()

    def gather(sl, b):
        pltpu.async_copy(y_hbm.at[src_v.at[sl]], rows_v.at[b], gsem.at[b])

    def drain_gather(sl, b):
        pltpu.make_async_copy(y_hbm.at[src_v.at[sl]], rows_v.at[b],
                              gsem.at[b]).wait()

    def scatter(sl, b):
        pltpu.async_copy(rows_v.at[b], acc_sh.at[dst_v.at[sl]], ssem.at[b],
                         add=True)

    def drain_scatter(sl, b):
        pltpu.make_async_copy(rows_v.at[b], acc_sh.at[dst_v.at[sl]],
                              ssem.at[b]).wait()

    zb = rows_v.at[0]
    _zero_vmem_rows(zb, C, D)
    base = s * RPT
    for r in range(RPT // C):
        pltpu.sync_copy(zb, acc_sh.at[pl.ds(base + r * C, C)])
    for g0 in range(SLOTS - 2):
        src_dma(g0, g0)
        dst_dma(g0, g0)
    wait_src(0, 0)
    gather(0, 0)
    plsc.subcore_barrier()

    t_last = trips - 1

    def outer(t, _):
        for u in range(SLOTS):
            g = t * SLOTS + u
            b = u % NBUF
            bn = (u + 1) % NBUF
            sl1 = (u + 1) % SLOTS
            sl6 = (u + 6) % SLOTS
            sl7 = (u + 7) % SLOTS

            if u == 0:
                @pl.when(t > 0)
                def _():
                    drain_scatter(sl7, bn)
            else:
                drain_scatter(sl7, bn)
            if u < 2:
                src_dma(g + 6, sl6)
                dst_dma(g + 6, sl6)
            else:
                @pl.when(t < t_last)
                def _():
                    src_dma(g + 6, sl6)
                    dst_dma(g + 6, sl6)
            if u < SLOTS - 1:
                wait_src(g + 1, sl1)
                gather(sl1, bn)
            else:
                @pl.when(t < t_last)
                def _():
                    wait_src(g + 1, sl1)
                    gather(sl1, bn)
            drain_gather(u, b)
            wait_dst(g, u)
            scatter(u, b)
        return 0

    lax.fori_loop(0, trips, outer, 0)
    drain_scatter((NCH - 1) % SLOTS, (NCH - 1) % NBUF)
    plsc.subcore_barrier()
    pltpu.sync_copy(acc_sh.at[pl.ds(s * RPT, RPT)],
                    out_hbm.at[c, pl.ds(s * RPT, RPT)])


BM = 1024
GRID = (N + BM - 1) // BM


def _dinv_of(degp_ref):
    deg = degp_ref[0, :, 0:1] + degp_ref[1, :, 0:1] + 1.0
    return lax.rsqrt(deg)


def _pre_body(x_ref, wp_ref, bp_ref, wc0_ref, degp_ref, h_ref, y_ref):
    h = jnp.dot(x_ref[...], wp_ref[...],
                preferred_element_type=jnp.float32) + bp_ref[...]
    h_ref[...] = h
    y_ref[...] = jnp.dot(h * _dinv_of(degp_ref), wc0_ref[...],
                         preferred_element_type=jnp.float32)


def _make_layer_body(residual, has_next):
    def body(parts_ref, y_ref, hprev_ref, degp_ref, bc_ref, g_ref, b_ref,
             *rest):
        if has_next:
            wc_ref, h_ref, ynext_ref = rest
        else:
            wc_ref = None
            (h_ref,) = rest
        dinv = _dinv_of(degp_ref)
        acc = parts_ref[0] + parts_ref[1] + y_ref[...]
        t = acc * dinv + bc_ref[...]
        mu = jnp.mean(t, axis=-1, keepdims=True)
        tc = t - mu
        var = jnp.mean(tc * tc, axis=-1, keepdims=True)
        t = tc * lax.rsqrt(var + 1e-5) * g_ref[...] + b_ref[...]
        t = jnp.maximum(t, 0.0)
        if residual:
            t = t + hprev_ref[...]
        h_ref[...] = t
        if has_next:
            ynext_ref[...] = jnp.dot(t * dinv, wc_ref[...],
                                     preferred_element_type=jnp.float32)

    return body


def _row_spec():
    return pl.BlockSpec((BM, D), lambda i: (i, 0))


def _full_spec(shape):
    nd = len(shape)
    return pl.BlockSpec(shape, lambda i: (0,) * nd)


def _degp_spec():
    return pl.BlockSpec((2, BM, D), lambda i: (0, i, 0))


def _tc_pre(x, Wp, bp, Wc0, degp):
    return pl.pallas_call(
        _pre_body,
        grid=(GRID,),
        in_specs=[
            _row_spec(),
            _full_spec((D, D)),
            _full_spec((1, D)),
            _full_spec((D, D)),
            _degp_spec(),
        ],
        out_specs=[_row_spec(), _row_spec()],
        out_shape=[
            jax.ShapeDtypeStruct((N, D), jnp.float32),
            jax.ShapeDtypeStruct((N, D), jnp.float32),
        ],
    )(x, Wp, bp.reshape(1, D), Wc0, degp)


def _tc_layer(parts, y, hprev, degp, bc_i, g_i, b_i, wc_next, residual):
    has_next = wc_next is not None
    ins = [parts, y, hprev, degp, bc_i.reshape(1, D), g_i.reshape(1, D),
           b_i.reshape(1, D)]
    in_specs = [
        pl.BlockSpec((2, BM, D), lambda i: (0, i, 0)),
        _row_spec(),
        _row_spec(),
        _degp_spec(),
        _full_spec((1, D)),
        _full_spec((1, D)),
        _full_spec((1, D)),
    ]
    if has_next:
        ins.append(wc_next)
        in_specs.append(_full_spec((D, D)))
        out_specs = [_row_spec(), _row_spec()]
        out_shape = [
            jax.ShapeDtypeStruct((N, D), jnp.float32),
            jax.ShapeDtypeStruct((N, D), jnp.float32),
        ]
    else:
        out_specs = [_row_spec()]
        out_shape = [jax.ShapeDtypeStruct((N, D), jnp.float32)]
    res = pl.pallas_call(
        _make_layer_body(residual, has_next),
        grid=(GRID,),
        in_specs=in_specs,
        out_specs=out_specs,
        out_shape=out_shape,
    )(*ins)
    return res if has_next else (res[0], None)


def kernel(x, edge_index, Wp, bp, Wc, bc, gamma, beta):
    src = edge_index[0]
    dst = edge_index[1]
    pad = PE - E
    pad_iota = jnp.arange(pad, dtype=jnp.int32)
    pad_src = pad_iota % N
    pad_dst = N + pad_iota % (NPAD - N)
    srcp = jnp.concatenate([src, pad_src])
    dstp = jnp.concatenate([dst, pad_dst])
    srcp = srcp.reshape(NW, NCH, C)
    dstp = dstp.reshape(NW, NCH, C)

    degp = _sc_degree(dstp)
    h, y = _tc_pre(x, Wp, bp, Wc[0], degp)
    for i in range(L):
        parts = _sc_scatter(y, srcp, dstp)
        wc_next = Wc[i + 1] if i < L - 1 else None
        h, y = _tc_layer(parts, y, h, degp, bc[i], gamma[i], beta[i],
                         wc_next, residual=(i > 0))
    return h


# speedup vs baseline: 3.6426x; 1.0897x over previous
import functools

import jax
import jax.numpy as jnp
from jax import lax
from jax.experimental import pallas as pl
from jax.experimental.pallas import tpu as pltpu
from jax.experimental.pallas import tpu_sc as plsc

N = 10000
E = 320000
D = 128
L = 3

NC = 2
NS = 16
NW = NC * NS
C = 120
NCH = 90
PE = NW * NCH * C
NBUF = 3
SLOTS = 6
DW = 32
NPAD = 10240
RPT = NPAD // NS


_MESH = plsc.VectorSubcoreMesh(core_axis_name="c", subcore_axis_name="s")


def _zero_vmem_rows(buf, nrows, ncols):
    z = jnp.zeros((16,), jnp.float32)

    def row(i, _):
        for j in range(ncols // 16):
            buf[i, pl.ds(j * 16, 16)] = z
        return 0

    lax.fori_loop(0, nrows, row, 0)


@functools.partial(
    pl.kernel,
    out_type=jax.ShapeDtypeStruct((NC, NPAD, DW), jnp.float32),
    mesh=_MESH,
    scratch_types=[
        pltpu.VMEM((NCH, C), jnp.int32),
        pltpu.VMEM((C, DW), jnp.float32),
        pltpu.VMEM_SHARED((NPAD, DW), jnp.float32),
        pltpu.SemaphoreType.DMA,
    ],
)
def _sc_degree(dst_hbm, out_hbm, dst_v, ones_v, deg_sh, sem):
    c = lax.axis_index("c")
    s = lax.axis_index("s")
    w = c * NS + s
    pltpu.sync_copy(dst_hbm.at[w], dst_v)
    _zero_vmem_rows(ones_v, C, DW)
    base = s * RPT
    nfull = RPT // C
    for r in range(nfull):
        pltpu.sync_copy(ones_v, deg_sh.at[pl.ds(base + r * C, C)])
    rem = RPT - nfull * C
    if rem:
        pltpu.sync_copy(ones_v.at[pl.ds(0, rem)],
                        deg_sh.at[pl.ds(base + nfull * C, rem)])
    one = jnp.ones((16,), jnp.float32)

    def fill(i, _):
        for j in range(DW // 16):
            ones_v[i, pl.ds(j * 16, 16)] = one
        return 0

    lax.fori_loop(0, C, fill, 0)
    plsc.subcore_barrier()

    def chunk(g, _):
        pltpu.async_copy(ones_v, deg_sh.at[dst_v.at[g]], sem, add=True)
        return 0

    lax.fori_loop(0, NCH, chunk, 0)

    def drain(g, _):
        pltpu.make_async_copy(ones_v, deg_sh.at[dst_v.at[g]], sem).wait()
        return 0

    lax.fori_loop(0, NCH, drain, 0)
    plsc.subcore_barrier()
    pltpu.sync_copy(deg_sh.at[pl.ds(s * RPT, RPT)],
                    out_hbm.at[c, pl.ds(s * RPT, RPT)])


@functools.partial(
    pl.kernel,
    out_type=jax.ShapeDtypeStruct((NC, NPAD, D), jnp.float32),
    mesh=_MESH,
    scratch_types=[
        pltpu.VMEM((SLOTS, C), jnp.int32),
        pltpu.VMEM((SLOTS, C), jnp.int32),
        pltpu.VMEM((NBUF, C, D), jnp.float32),
        pltpu.VMEM_SHARED((NPAD, D), jnp.float32),
        pltpu.SemaphoreType.DMA((SLOTS,)),
        pltpu.SemaphoreType.DMA((SLOTS,)),
        pltpu.SemaphoreType.DMA((NBUF,)),
        pltpu.SemaphoreType.DMA((NBUF,)),
        pltpu.SemaphoreType.DMA,
    ],
)
def _sc_scatter(y_hbm, src_hbm, dst_hbm, out_hbm, src_v, dst_v, rows_v,
                acc_sh, isem, dsem, gsem, ssem, zsem):
    c = lax.axis_index("c")
    s = lax.axis_index("s")
    w = c * NS + s
    trips = NCH // SLOTS

    def src_dma(g, sl):
        pltpu.async_copy(src_hbm.at[w, g], src_v.at[sl], isem.at[sl])

    def wait_src(g, sl):
        pltpu.make_async_copy(src_hbm.at[w, g], src_v.at[sl],
                              isem.at[sl]).wait()

    def dst_dma(g, sl):
        pltpu.async_copy(dst_hbm.at[w, g], dst_v.at[sl], dsem.at[sl])

    def wait_dst(g, sl):
        pltpu.make_async_copy(dst_hbm.at[w, g], dst_v.at[sl],
                              dsem.at[sl]).wait()

    def gather(sl, b):
        pltpu.async_copy(y_hbm.at[src_v.at[sl]], rows_v.at[b], gsem.at[b])

    def drain_gather(sl, b):
        pltpu.make_async_copy(y_hbm.at[src_v.at[sl]], rows_v.at[b],
                              gsem.at[b]).wait()

    def scatter(sl, b):
        pltpu.async_copy(rows_v.at[b], acc_sh.at[dst_v.at[sl]], ssem.at[b],
                         add=True)

    def drain_scatter(sl, b):
        pltpu.make_async_copy(rows_v.at[b], acc_sh.at[dst_v.at[sl]],
                              ssem.at[b]).wait()

    zb = rows_v.at[0]
    _zero_vmem_rows(zb, C, D)
    base = s * RPT
    nfull = RPT // C
    rem = RPT - nfull * C
    for r in range(nfull):
        pltpu.async_copy(zb, acc_sh.at[pl.ds(base + r * C, C)], zsem)
    if rem:
        pltpu.async_copy(zb.at[pl.ds(0, rem)],
                         acc_sh.at[pl.ds(base + nfull * C, rem)], zsem)
    for g0 in range(SLOTS - 2):
        src_dma(g0, g0)
        dst_dma(g0, g0)
    for r in range(nfull):
        pltpu.make_async_copy(zb, acc_sh.at[pl.ds(base + r * C, C)],
                              zsem).wait()
    if rem:
        pltpu.make_async_copy(zb.at[pl.ds(0, rem)],
                              acc_sh.at[pl.ds(base + nfull * C, rem)],
                              zsem).wait()
    wait_src(0, 0)
    gather(0, 0)
    plsc.subcore_barrier()

    t_last = trips - 1

    def outer(t, _):
        for u in range(SLOTS):
            g = t * SLOTS + u
            b = u % NBUF
            bn = (u + 1) % NBUF
            bp = (u + NBUF - 1) % NBUF
            sl1 = (u + 1) % SLOTS
            sl4 = (u + 4) % SLOTS
            slp = (u + SLOTS - 1) % SLOTS

            if u <= 1:
                @pl.when(t > 0)
                def _():
                    drain_scatter((u + SLOTS - 2) % SLOTS, bn)
            else:
                drain_scatter((u + SLOTS - 2) % SLOTS, bn)
            if u < 2:
                src_dma(g + 4, sl4)
                dst_dma(g + 4, sl4)
            else:
                @pl.when(t < t_last)
                def _():
                    src_dma(g + 4, sl4)
                    dst_dma(g + 4, sl4)
            if u < SLOTS - 1:
                wait_src(g + 1, sl1)
                gather(sl1, bn)
            else:
                @pl.when(t < t_last)
                def _():
                    wait_src(g + 1, sl1)
                    gather(sl1, bn)
            drain_gather(u, b)
            wait_dst(g, u)
            scatter(u, b)
        return 0

    lax.fori_loop(0, trips, outer, 0)
    drain_scatter((NCH - 2) % SLOTS, (NCH - 2) % NBUF)
    drain_scatter((NCH - 1) % SLOTS, (NCH - 1) % NBUF)
    plsc.subcore_barrier()
    pltpu.sync_copy(acc_sh.at[pl.ds(s * RPT, RPT)],
                    out_hbm.at[c, pl.ds(s * RPT, RPT)])


BM = 1024
GRID = (N + BM - 1) // BM


def _dinv_of(degp_ref):
    deg = degp_ref[0, :, 0:1] + degp_ref[1, :, 0:1] + 1.0
    return lax.rsqrt(deg)


def _pre_body(x_ref, wp_ref, bp_ref, wc0_ref, degp_ref, h_ref, y_ref):
    h = jnp.dot(x_ref[...], wp_ref[...],
                preferred_element_type=jnp.float32) + bp_ref[...]
    h_ref[...] = h
    y_ref[...] = jnp.dot(h * _dinv_of(degp_ref), wc0_ref[...],
                         preferred_element_type=jnp.float32)


def _make_layer_body(residual, has_next):
    def body(parts_ref, y_ref, hprev_ref, degp_ref, bc_ref, g_ref, b_ref,
             *rest):
        if has_next:
            wc_ref, h_ref, ynext_ref = rest
        else:
            wc_ref = None
            (h_ref,) = rest
        dinv = _dinv_of(degp_ref)
        acc = parts_ref[0] + parts_ref[1] + y_ref[...]
        t = acc * dinv + bc_ref[...]
        mu = jnp.mean(t, axis=-1, keepdims=True)
        tc = t - mu
        var = jnp.mean(tc * tc, axis=-1, keepdims=True)
        t = tc * lax.rsqrt(var + 1e-5) * g_ref[...] + b_ref[...]
        t = jnp.maximum(t, 0.0)
        if residual:
            t = t + hprev_ref[...]
        h_ref[...] = t
        if has_next:
            ynext_ref[...] = jnp.dot(t * dinv, wc_ref[...],
                                     preferred_element_type=jnp.float32)

    return body


def _row_spec():
    return pl.BlockSpec((BM, D), lambda i: (i, 0))


def _full_spec(shape):
    nd = len(shape)
    return pl.BlockSpec(shape, lambda i: (0,) * nd)


def _degp_spec():
    return pl.BlockSpec((2, BM, DW), lambda i: (0, i, 0))


def _tc_pre(x, Wp, bp, Wc0, degp):
    return pl.pallas_call(
        _pre_body,
        grid=(GRID,),
        in_specs=[
            _row_spec(),
            _full_spec((D, D)),
            _full_spec((1, D)),
            _full_spec((D, D)),
            _degp_spec(),
        ],
        out_specs=[_row_spec(), _row_spec()],
        out_shape=[
            jax.ShapeDtypeStruct((N, D), jnp.float32),
            jax.ShapeDtypeStruct((N, D), jnp.float32),
        ],
    )(x, Wp, bp.reshape(1, D), Wc0, degp)


def _tc_layer(parts, y, hprev, degp, bc_i, g_i, b_i, wc_next, residual):
    has_next = wc_next is not None
    ins = [parts, y, hprev, degp, bc_i.reshape(1, D), g_i.reshape(1, D),
           b_i.reshape(1, D)]
    in_specs = [
        pl.BlockSpec((2, BM, D), lambda i: (0, i, 0)),
        _row_spec(),
        _row_spec(),
        _degp_spec(),
        _full_spec((1, D)),
        _full_spec((1, D)),
        _full_spec((1, D)),
    ]
    if has_next:
        ins.append(wc_next)
        in_specs.append(_full_spec((D, D)))
        out_specs = [_row_spec(), _row_spec()]
        out_shape = [
            jax.ShapeDtypeStruct((N, D), jnp.float32),
            jax.ShapeDtypeStruct((N, D), jnp.float32),
        ]
    else:
        out_specs = [_row_spec()]
        out_shape = [jax.ShapeDtypeStruct((N, D), jnp.float32)]
    res = pl.pallas_call(
        _make_layer_body(residual, has_next),
        grid=(GRID,),
        in_specs=in_specs,
        out_specs=out_specs,
        out_shape=out_shape,
    )(*ins)
    return res if has_next else (res[0], None)


def kernel(x, edge_index, Wp, bp, Wc, bc, gamma, beta):
    src = edge_index[0]
    dst = edge_index[1]
    pad = PE - E
    pad_iota = jnp.arange(pad, dtype=jnp.int32)
    pad_src = pad_iota % N
    pad_dst = N + pad_iota % (NPAD - N)
    srcp = jnp.concatenate([src, pad_src])
    dstp = jnp.concatenate([dst, pad_dst])
    srcp = srcp.reshape(NW, NCH, C)
    dstp = dstp.reshape(NW, NCH, C)

    degp = _sc_degree(dstp)
    h, y = _tc_pre(x, Wp, bp, Wc[0], degp)
    for i in range(L):
        parts = _sc_scatter(y, srcp, dstp)
        wc_next = Wc[i + 1] if i < L - 1 else None
        h, y = _tc_layer(parts, y, h, degp, bc[i], gamma[i], beta[i],
                         wc_next, residual=(i > 0))
    return h
